# baseline (speedup 1.0000x reference)
import jax
import jax.numpy as jnp
from jax import lax
from jax.experimental import pallas as pl
from jax.experimental.pallas import tpu as pltpu
from jax.experimental.pallas import tpu_sc as plsc

_VOCAB = 1_000_000
_D = 64
_N = 4096
_U = 200
_NC = 2
_NS = 16
_NW = _NC * _NS
_SEQ_PER_W = _N // _NW
_L = 16
_KV = _D // _L
_C1 = 128
_C2 = _U - _C1
_UNROLL = 8
_GDEPTH = 4
_SDEPTH = 2


def _sc_decoder(y_hbm, table_hbm, w0_hbm, w1_hbm, out_hbm,
                idx_v, rows0, rows1, rows2, rows3, out0, out1, w0_v, w1_v,
                gsem0, gsem1, gsem2, gsem3, ssem0, ssem1):
    wid = lax.axis_index("s") * _NC + lax.axis_index("c")
    wbase = wid * _SEQ_PER_W * _U
    pltpu.sync_copy(w0_hbm, w0_v)
    pltpu.sync_copy(w1_hbm, w1_v)
    pltpu.sync_copy(y_hbm.at[pl.ds(wbase, _SEQ_PER_W * _U)], idx_v)
    w0r = [w0_v[pl.ds(_L * k, _L)] for k in range(_KV)]
    w1r = [w1_v[pl.ds(_L * k, _L)] for k in range(_KV)]
    zero = jnp.zeros((_L,), jnp.float32)
    rows = (rows0, rows1, rows2, rows3)
    outs = (out0, out1)
    gsems = (gsem0, gsem1, gsem2, gsem3)
    ssems = (ssem0, ssem1)

    def fire_gather(j, g):
        off = j * _U
        pltpu.async_copy(table_hbm.at[idx_v.at[pl.ds(off, _C1)]],
                         rows[g].at[pl.ds(0, _C1)], gsems[g])
        pltpu.async_copy(table_hbm.at[idx_v.at[pl.ds(off + _C1, _C2)]],
                         rows[g].at[pl.ds(_C1, _C2)], gsems[g])

    def wait_gather(g):
        pltpu.make_async_copy(table_hbm.at[idx_v.at[pl.ds(0, _C1)]],
                              rows[g].at[pl.ds(0, _C1)], gsems[g]).wait()
        pltpu.make_async_copy(table_hbm.at[idx_v.at[pl.ds(_C1, _C2)]],
                              rows[g].at[pl.ds(_C1, _C2)], gsems[g]).wait()

    def compute(g, p):
        def row_block(ib, prev):
            cur = prev
            i0 = ib * _UNROLL
            for r in range(_UNROLL):
                nxt = []
                for k in range(_KV):
                    c = rows[g][i0 + r, pl.ds(_L * k, _L)]
                    outs[p][i0 + r, pl.ds(_L * k, _L)] = jnp.maximum(
                        c * w1r[k] + cur[k] * w0r[k], 0.0)
                    nxt.append(c)
                cur = nxt
            return tuple(cur)
        lax.fori_loop(0, _U // _UNROLL, row_block, (zero,) * _KV)

    def fire_store(j, p):
        pltpu.async_copy(outs[p], out_hbm.at[pl.ds(wbase + j * _U, _U)],
                         ssems[p])

    def wait_store(p):
        pltpu.make_async_copy(outs[p], out_hbm.at[pl.ds(wbase, _U)],
                              ssems[p]).wait()

    for j in range(_GDEPTH):
        fire_gather(j, j)

    def step(j, g, p):
        wait_gather(g)

        @pl.when(j >= _SDEPTH)
        def _():
            wait_store(p)

        compute(g, p)
        fire_store(j, p)

        @pl.when(j + _GDEPTH < _SEQ_PER_W)
        def _():
            fire_gather(j + _GDEPTH, g)

    def block_body(jj, carry):
        for r in range(_GDEPTH):
            step(_GDEPTH * jj + r, r, r % _SDEPTH)
        return carry

    lax.fori_loop(0, _SEQ_PER_W // _GDEPTH, block_body, 0)
    wait_store(0)
    wait_store(1)


def kernel(y, emb_weight, conv_weight):
    assert y.shape == (_N, _U) and emb_weight.shape == (_VOCAB, _D)
    y_idx = jnp.clip(y, 0, _VOCAB - 1).astype(jnp.int32).reshape(_N * _U)
    w0 = conv_weight[:, 0, 0]
    w1 = conv_weight[:, 0, 1]
    mesh = plsc.VectorSubcoreMesh(core_axis_name="c", subcore_axis_name="s")
    f = pl.kernel(
        _sc_decoder,
        mesh=mesh,
        compiler_params=pltpu.CompilerParams(use_tc_tiling_on_sc=False),
        out_type=jax.ShapeDtypeStruct((_N * _U, _D), jnp.float32),
        scratch_types=[
            pltpu.VMEM((_SEQ_PER_W * _U,), jnp.int32),
            pltpu.VMEM((_U, _D), jnp.float32),
            pltpu.VMEM((_U, _D), jnp.float32),
            pltpu.VMEM((_U, _D), jnp.float32),
            pltpu.VMEM((_U, _D), jnp.float32),
            pltpu.VMEM((_U, _D), jnp.float32),
            pltpu.VMEM((_U, _D), jnp.float32),
            pltpu.VMEM((_D,), jnp.float32),
            pltpu.VMEM((_D,), jnp.float32),
            pltpu.SemaphoreType.DMA,
            pltpu.SemaphoreType.DMA,
            pltpu.SemaphoreType.DMA,
            pltpu.SemaphoreType.DMA,
            pltpu.SemaphoreType.DMA,
            pltpu.SemaphoreType.DMA,
        ],
    )
    out = f(y_idx, emb_weight, w0, w1)
    return out.reshape(_N, _U, _D)
